# Initial kernel scaffold; baseline (speedup 1.0000x reference)
import jax
import jax.numpy as jnp
from jax.experimental import pallas as pl


def kernel(fixed_values, refinable_params, refinable_idx):
    raise NotImplementedError("write your pallas kernel here")


# speedup vs baseline: 2.2501x; 2.2501x over previous
import functools

import jax
import jax.numpy as jnp
from jax import lax
from jax.experimental import pallas as pl
from jax.experimental.pallas import tpu as pltpu
from jax.experimental.pallas import tpu_sc as plsc

_ROWS, _COLS = 16384, 1024
_N = _ROWS * _COLS
_R = _N // 4

_NC, _NS = 2, 16
_NW = _NC * _NS
_B = 128
_K = 16
_RPW = _R // (_B * _NW)
_G = _RPW // _K


_COPY_BLOCK = 1024


def _copy_body(src_ref, dst_ref):
    dst_ref[...] = src_ref[...]


def _tc_copy(x):
    grid = _ROWS // _COPY_BLOCK
    return pl.pallas_call(
        _copy_body,
        grid=(grid,),
        in_specs=[pl.BlockSpec((_COPY_BLOCK, _COLS), lambda i: (i, 0))],
        out_specs=pl.BlockSpec((_COPY_BLOCK, _COLS), lambda i: (i, 0)),
        out_shape=jax.ShapeDtypeStruct((_ROWS, _COLS), jnp.float32),
    )(x)


def _sc_scatter_body(out_ref, idx_ref, par_ref, idx_v, par_v, sem):
    c = lax.axis_index("c")
    s = lax.axis_index("s")
    wid = s * _NC + c
    row0 = wid * _RPW

    def group(g, carry):
        r = row0 + g * _K
        pltpu.sync_copy(idx_ref.at[pl.ds(r, _K), :], idx_v)
        pltpu.sync_copy(par_ref.at[pl.ds(r, _K), :], par_v)
        copies = [
            pltpu.async_copy(par_v.at[j], out_ref.at[idx_v.at[j]], sem)
            for j in range(_K)
        ]
        for cp in copies:
            cp.wait()
        return carry

    lax.fori_loop(0, _G, group, None)


def _make_sc_scatter():
    mesh = plsc.VectorSubcoreMesh(
        core_axis_name="c", subcore_axis_name="s",
        num_cores=_NC, num_subcores=_NS,
    )
    return pl.kernel(
        _sc_scatter_body,
        out_type=(),
        mesh=mesh,
        scratch_types=[
            pltpu.VMEM((_K, _B), jnp.int32),
            pltpu.VMEM((_K, _B), jnp.float32),
            pltpu.SemaphoreType.DMA,
        ],
    )


def kernel(fixed_values, refinable_params, refinable_idx):
    idx2 = refinable_idx.astype(jnp.int32).reshape(_R // _B, _B)
    par2 = refinable_params.reshape(_R // _B, _B)
    out = _tc_copy(fixed_values)
    out_ref = jax.new_ref(out.reshape(_N))
    _make_sc_scatter()(out_ref, idx2, par2)
    return out_ref[...].reshape(_ROWS, _COLS)
